# baseline (speedup 1.0000x reference)
import jax
import jax.numpy as jnp
from jax import lax
from jax.experimental import pallas as pl
from jax.experimental.pallas import tpu as pltpu
from jax.experimental.pallas import tpu_sc as plsc

N_NODES = 10000
N_MSG = 320000
D = 128
K = 128
L = 16
NC = 2
NS = 16
NCHUNK = 2 * NS
NPC = N_NODES // NC
TPC = N_MSG // NCHUNK
B = 128
NBLK = (TPC + B - 1) // B
RPT = 320
ACC_ROWS = NS * RPT
DUMMY = NPC + 8
SENTINEL = 2 ** 30
FLUSH_ROWS = 80


def _sc_body(ids_hbm, msgs_hbm, ts_hbm, psums, pcnt, pts,
             msg_buf0, msg_buf1,
             idx_buf0, idx_buf1, idx_buf2, idx_buf3,
             ts_st0, ts_st1, ts_st2, ts_st3,
             one_st, ids_buf, ts_buf, zbuf_v, zbuf_v2,
             acc_m, acc_c, acc_t, ld_sem, sct_sem, aux_sem):
    c = lax.axis_index("c")
    s = lax.axis_index("s")
    lo = c * NPC
    zero16 = jnp.zeros((L,), jnp.float32)
    iota16 = lax.iota(jnp.int32, L)

    def _z_m(q, _):
        msg_buf0[q // 8, pl.ds((q % 8) * L, L)] = zero16
        return 0
    lax.fori_loop(0, 2 * FLUSH_ROWS * 8, _z_m, 0)
    zsrc = msg_buf0.at[pl.ds(0, 2 * FLUSH_ROWS)]

    def _z_v(q, _):
        zbuf_v[pl.ds(q * L, L)] = zero16
        return 0
    lax.fori_loop(0, RPT // L, _z_v, 0)

    def _one(q, _):
        one_st[pl.ds(q * L, L)] = zero16 + 1.0
        return 0
    lax.fori_loop(0, B // L, _one, 0)

    row0 = s * RPT
    zdscs = [pltpu.async_copy(zsrc,
                              acc_m.at[pl.ds(row0 + k * 2 * FLUSH_ROWS,
                                             2 * FLUSH_ROWS)], aux_sem)
             for k in range(RPT // (2 * FLUSH_ROWS))]
    zdscs.append(pltpu.async_copy(zbuf_v, acc_c.at[pl.ds(row0, RPT)],
                                  aux_sem))
    zdscs.append(pltpu.async_copy(zbuf_v, acc_t.at[pl.ds(row0, RPT)],
                                  aux_sem))
    for d in zdscs:
        d.wait()

    plsc.subcore_barrier()

    def _process_chunk(chunk):
        base = chunk * TPC
        sa = pltpu.async_copy(ids_hbm.at[pl.ds(base, TPC + K)], ids_buf,
                              aux_sem)
        sb = pltpu.async_copy(ts_hbm.at[pl.ds(base, TPC)], ts_buf, aux_sem)
        sa.wait()
        sb.wait()

        def _scan(g, carry):
            blk_lo, blk_hi = carry
            bidx = g * L + iota16
            off = jnp.minimum(bidx * B, TPC - B)
            first = plsc.load_gather(ids_buf, [off])
            last_id = plsc.load_gather(ids_buf, [off + B - 1])
            hit = (first < lo + NPC) & (last_id >= lo) & (bidx < NBLK)
            lo_cand = jnp.min(jnp.where(hit, bidx, NBLK))
            hi_cand = jnp.max(jnp.where(hit, bidx + 1, 0))
            return (jnp.minimum(blk_lo, lo_cand),
                    jnp.maximum(blk_hi, hi_cand))
        blk_lo, blk_hi = lax.fori_loop(0, (NBLK + L - 1) // L, _scan,
                                       (jnp.int32(NBLK), jnp.int32(0)))

        def _compute_idx(ls, roff, idx_buf, ts_st):
            def _blk(j, _):
                r = roff + j * L
                pos = r + iota16
                ids_c = ids_buf[pl.ds(r, L)]
                ids_k = plsc.load_gather(ids_buf, [pos + K])
                ids_n = plsc.load_gather(ids_buf, [pos + 1])
                local = ids_c - lo
                valid = pos >= ls
                take = ((ids_k != ids_c) & (local >= 0) & (local < NPC)
                        & valid)
                lastm = (ids_n != ids_c) & valid
                idx_buf[pl.ds(j * L, L)] = jnp.where(take, local, DUMMY)
                ts_st[pl.ds(j * L, L)] = jnp.where(
                    lastm, ts_buf[pl.ds(r, L)], 0.0)
                return 0
            lax.fori_loop(0, B // L, _blk, 0)

        row_start = blk_lo * B

        def _soff(sb):
            return jnp.minimum(row_start + sb * 2 * B, TPC - 2 * B)

        def _start_load(sb, msg_buf):
            return pltpu.async_copy(
                msgs_hbm.at[pl.ds(base + _soff(sb), 2 * B)], msg_buf, ld_sem)

        def _start_scatter(msg_buf, idx_buf, ts_st):
            d0 = pltpu.async_copy(msg_buf, acc_m.at[idx_buf], sct_sem,
                                  add=True)
            d1 = pltpu.async_copy(one_st, acc_c.at[idx_buf], sct_sem,
                                  add=True)
            d2 = pltpu.async_copy(ts_st, acc_t.at[idx_buf], sct_sem,
                                  add=True)
            return d0, d1, d2

        def _drain(dscs):
            for d in dscs:
                d.wait()

        span = jnp.maximum(blk_hi - blk_lo, 0)
        nsb2 = ((span + 1) // 2 + 1) // 2
        la = _start_load(0, msg_buf0)
        lb = _start_load(1, msg_buf1)

        def _p2(p, _):
            sb0 = 2 * p
            sb1 = 2 * p + 1
            la.wait()
            t0 = jnp.minimum(row_start + sb0 * 2 * B, TPC)
            _compute_idx(t0, _soff(sb0), idx_buf0, ts_st0)
            s00 = _start_scatter(msg_buf0.at[pl.ds(0, B)], idx_buf0, ts_st0)
            _compute_idx(t0, _soff(sb0) + B, idx_buf1, ts_st1)
            s01 = _start_scatter(msg_buf0.at[pl.ds(B, B)], idx_buf1, ts_st1)
            lb.wait()
            t1 = jnp.minimum(row_start + sb1 * 2 * B, TPC)
            _compute_idx(t1, _soff(sb1), idx_buf2, ts_st2)
            s10 = _start_scatter(msg_buf1.at[pl.ds(0, B)], idx_buf2, ts_st2)
            _compute_idx(t1, _soff(sb1) + B, idx_buf3, ts_st3)
            s11 = _start_scatter(msg_buf1.at[pl.ds(B, B)], idx_buf3, ts_st3)
            _drain(s00)
            _drain(s01)
            _start_load(sb0 + 2, msg_buf0)
            _drain(s10)
            _drain(s11)
            _start_load(sb1 + 2, msg_buf1)
            return 0

        lax.fori_loop(0, nsb2, _p2, 0)
        la.wait()
        lb.wait()

    _process_chunk(s)
    _process_chunk(NCHUNK - 1 - s)

    plsc.subcore_barrier()

    def _mrow(k):
        return pl.ds(row0 + k * FLUSH_ROWS, FLUSH_ROWS)

    def _hrow(k):
        return pl.ds(row0 + k * FLUSH_ROWS, FLUSH_ROWS)

    FR2 = 2 * FLUSH_ROWS
    mb = [msg_buf0.at[pl.ds(0, FR2)], msg_buf1.at[pl.ds(0, FR2)]]

    def _row2(k):
        return pl.ds(row0 + k * FR2, FR2)

    d0 = pltpu.async_copy(acc_m.at[_row2(0)], mb[0], aux_sem)
    d1 = pltpu.async_copy(acc_m.at[_row2(1)], mb[1], aux_sem)
    dv0 = pltpu.async_copy(acc_c.at[pl.ds(row0, RPT)], zbuf_v, aux_sem)
    dv1 = pltpu.async_copy(acc_t.at[pl.ds(row0, RPT)], zbuf_v2, aux_sem)
    d0.wait()
    h0 = pltpu.async_copy(mb[0], psums.at[c, _row2(0)], sct_sem)
    d1.wait()
    h1 = pltpu.async_copy(mb[1], psums.at[c, _row2(1)], sct_sem)
    dv0.wait()
    hv0 = pltpu.async_copy(zbuf_v, pcnt.at[pl.ds(c * ACC_ROWS + row0, RPT)],
                           sct_sem)
    dv1.wait()
    hv1 = pltpu.async_copy(zbuf_v2, pts.at[pl.ds(c * ACC_ROWS + row0, RPT)],
                           sct_sem)
    h0.wait()
    h1.wait()
    hv0.wait()
    hv1.wait()


def _sc_aggregate(ids_pad, messages, timestamps):
    mesh = plsc.VectorSubcoreMesh(core_axis_name="c", subcore_axis_name="s")
    return pl.kernel(
        _sc_body,
        out_type=[
            jax.ShapeDtypeStruct((NC, ACC_ROWS, D), jnp.float32),
            jax.ShapeDtypeStruct((NC * ACC_ROWS,), jnp.float32),
            jax.ShapeDtypeStruct((NC * ACC_ROWS,), jnp.float32),
        ],
        mesh=mesh,
        scratch_types=[
            pltpu.VMEM((2 * B, D), jnp.float32),
            pltpu.VMEM((2 * B, D), jnp.float32),
            pltpu.VMEM((B,), jnp.int32),
            pltpu.VMEM((B,), jnp.int32),
            pltpu.VMEM((B,), jnp.int32),
            pltpu.VMEM((B,), jnp.int32),
            pltpu.VMEM((B,), jnp.float32),
            pltpu.VMEM((B,), jnp.float32),
            pltpu.VMEM((B,), jnp.float32),
            pltpu.VMEM((B,), jnp.float32),
            pltpu.VMEM((B,), jnp.float32),
            pltpu.VMEM((TPC + K,), jnp.int32),
            pltpu.VMEM((TPC,), jnp.float32),
            pltpu.VMEM((RPT,), jnp.float32),
            pltpu.VMEM((RPT,), jnp.float32),
            pltpu.VMEM_SHARED((ACC_ROWS, D), jnp.float32),
            pltpu.VMEM_SHARED((ACC_ROWS,), jnp.float32),
            pltpu.VMEM_SHARED((ACC_ROWS,), jnp.float32),
            pltpu.SemaphoreType.DMA,
            pltpu.SemaphoreType.DMA,
            pltpu.SemaphoreType.DMA,
        ],
        compiler_params=pltpu.CompilerParams(needs_layout_passes=False),
    )(ids_pad, messages, timestamps)


def _finalize_body(ps_ref, pc_ref, pt_ref, um_ref, ts_ref, msk_ref):
    sums = jnp.concatenate([ps_ref[0, :NPC, :], ps_ref[1, :NPC, :]], axis=0)
    cnt = jnp.concatenate([pc_ref[:NPC], pc_ref[ACC_ROWS:ACC_ROWS + NPC]],
                          axis=0)
    ts = jnp.concatenate([pt_ref[:NPC], pt_ref[ACC_ROWS:ACC_ROWS + NPC]],
                         axis=0)
    um_ref[...] = sums / jnp.maximum(cnt, 1.0)[:, None]
    ts_ref[...] = ts
    msk_ref[...] = (cnt > 0.0).astype(jnp.int32)


def _finalize(psums, pcnt, pts):
    return pl.pallas_call(
        _finalize_body,
        out_shape=[
            jax.ShapeDtypeStruct((N_NODES, D), jnp.float32),
            jax.ShapeDtypeStruct((N_NODES,), jnp.float32),
            jax.ShapeDtypeStruct((N_NODES,), jnp.int32),
        ],
    )(psums, pcnt, pts)


@jax.jit
def kernel(node_ids, messages, timestamps, memory):
    del memory
    ids = node_ids.astype(jnp.int32)
    ids_pad = jnp.concatenate([ids, jnp.full((K,), SENTINEL, jnp.int32)])
    psums, pcnt, pts = _sc_aggregate(ids_pad, messages, timestamps)
    um, ts, msk = _finalize(psums, pcnt, pts)
    return um, ts, msk.astype(bool)


# speedup vs baseline: 1.0241x; 1.0241x over previous
import jax
import jax.numpy as jnp
from jax import lax
from jax.experimental import pallas as pl
from jax.experimental.pallas import tpu as pltpu
from jax.experimental.pallas import tpu_sc as plsc

N_NODES = 10000
N_MSG = 320000
D = 128
K = 128
L = 16
NC = 2
NS = 16
NCHUNK = 2 * NS
NPC = N_NODES // NC
TPC = N_MSG // NCHUNK
B = 128
NBLK = (TPC + B - 1) // B
RPT = 320
ACC_ROWS = NS * RPT
DUMMY = NPC + 8
SENTINEL = 2 ** 30
FLUSH_ROWS = 80


def _sc_body(ids_hbm, msgs_hbm, ts_hbm, psums, pcnt, pts,
             msg_buf0, msg_buf1, msg_buf2,
             idx_buf0, idx_buf1, idx_buf2, ts_st0, ts_st1, ts_st2,
             one_st, ids_buf, ts_buf, zbuf_v, zbuf_v2,
             acc_m, acc_c, acc_t, ld_sem, sct_sem, aux_sem):
    c = lax.axis_index("c")
    s = lax.axis_index("s")
    lo = c * NPC
    zero16 = jnp.zeros((L,), jnp.float32)
    iota16 = lax.iota(jnp.int32, L)

    def _z_m(q, _):
        msg_buf0[q // 8, pl.ds((q % 8) * L, L)] = zero16
        return 0
    lax.fori_loop(0, FLUSH_ROWS * 8, _z_m, 0)
    zsrc = msg_buf0.at[pl.ds(0, FLUSH_ROWS)]

    def _z_v(q, _):
        zbuf_v[pl.ds(q * L, L)] = zero16
        return 0
    lax.fori_loop(0, RPT // L, _z_v, 0)

    def _one(q, _):
        one_st[pl.ds(q * L, L)] = zero16 + 1.0
        return 0
    lax.fori_loop(0, B // L, _one, 0)

    row0 = s * RPT
    zdscs = [pltpu.async_copy(zsrc,
                              acc_m.at[pl.ds(row0 + k * FLUSH_ROWS,
                                             FLUSH_ROWS)], aux_sem)
             for k in range(RPT // FLUSH_ROWS)]
    zdscs.append(pltpu.async_copy(zbuf_v, acc_c.at[pl.ds(row0, RPT)],
                                  aux_sem))
    zdscs.append(pltpu.async_copy(zbuf_v, acc_t.at[pl.ds(row0, RPT)],
                                  aux_sem))
    for d in zdscs:
        d.wait()

    plsc.subcore_barrier()

    def _process_chunk(chunk):
        base = chunk * TPC
        sa = pltpu.async_copy(ids_hbm.at[pl.ds(base, TPC + K)], ids_buf,
                              aux_sem)
        sb = pltpu.async_copy(ts_hbm.at[pl.ds(base, TPC)], ts_buf, aux_sem)
        sa.wait()
        sb.wait()

        def _scan(g, carry):
            blk_lo, blk_hi = carry
            bidx = g * L + iota16
            off = jnp.minimum(bidx * B, TPC - B)
            first = plsc.load_gather(ids_buf, [off])
            last_id = plsc.load_gather(ids_buf, [off + B - 1])
            hit = (first < lo + NPC) & (last_id >= lo) & (bidx < NBLK)
            lo_cand = jnp.min(jnp.where(hit, bidx, NBLK))
            hi_cand = jnp.max(jnp.where(hit, bidx + 1, 0))
            return (jnp.minimum(blk_lo, lo_cand),
                    jnp.maximum(blk_hi, hi_cand))
        blk_lo, blk_hi = lax.fori_loop(0, (NBLK + L - 1) // L, _scan,
                                       (jnp.int32(NBLK), jnp.int32(0)))

        def _compute_idx(bb, idx_buf, ts_st):
            off = jnp.minimum(bb * B, TPC - B)
            blk_valid = bb < blk_hi

            def _blk(j, _):
                r = off + j * L
                pos = r + iota16
                ids_c = ids_buf[pl.ds(r, L)]
                ids_k = plsc.load_gather(ids_buf, [pos + K])
                ids_n = plsc.load_gather(ids_buf, [pos + 1])
                local = ids_c - lo
                valid = (pos >= bb * B) & blk_valid
                take = ((ids_k != ids_c) & (local >= 0) & (local < NPC)
                        & valid)
                lastm = (ids_n != ids_c) & valid
                idx_buf[pl.ds(j * L, L)] = jnp.where(take, local, DUMMY)
                ts_st[pl.ds(j * L, L)] = jnp.where(
                    lastm, ts_buf[pl.ds(r, L)], 0.0)
                return 0
            lax.fori_loop(0, B // L, _blk, 0)

        def _start_load(bb, msg_buf):
            off = jnp.minimum(bb, NBLK - 1) * B
            off = jnp.minimum(off, TPC - B)
            return pltpu.async_copy(msgs_hbm.at[pl.ds(base + off, B)],
                                    msg_buf, ld_sem)

        def _start_scatter(msg_buf, idx_buf, ts_st):
            d0 = pltpu.async_copy(msg_buf, acc_m.at[idx_buf], sct_sem,
                                  add=True)
            d1 = pltpu.async_copy(one_st, acc_c.at[idx_buf], sct_sem,
                                  add=True)
            d2 = pltpu.async_copy(ts_st, acc_t.at[idx_buf], sct_sem,
                                  add=True)
            return d0, d1, d2

        def _drain(dscs):
            for d in dscs:
                d.wait()

        ntris = (jnp.maximum(blk_hi - blk_lo, 0) + 2) // 3
        la = _start_load(blk_lo, msg_buf0)
        lb = _start_load(blk_lo + 1, msg_buf1)
        lc = _start_load(blk_lo + 2, msg_buf2)

        def _tri(t, _):
            b = blk_lo + 3 * t
            la.wait()
            _compute_idx(b, idx_buf0, ts_st0)
            s0 = _start_scatter(msg_buf0, idx_buf0, ts_st0)
            lb.wait()
            _compute_idx(b + 1, idx_buf1, ts_st1)
            s1 = _start_scatter(msg_buf1, idx_buf1, ts_st1)
            _drain(s0)
            _start_load(b + 3, msg_buf0)
            lc.wait()
            _compute_idx(b + 2, idx_buf2, ts_st2)
            s2 = _start_scatter(msg_buf2, idx_buf2, ts_st2)
            _drain(s1)
            _start_load(b + 4, msg_buf1)
            _drain(s2)
            _start_load(b + 5, msg_buf2)
            return 0

        lax.fori_loop(0, ntris, _tri, 0)
        la.wait()
        lb.wait()
        lc.wait()

    _process_chunk(s)
    _process_chunk(NCHUNK - 1 - s)

    plsc.subcore_barrier()

    def _mrow(k):
        return pl.ds(row0 + k * FLUSH_ROWS, FLUSH_ROWS)

    def _hrow(k):
        return pl.ds(row0 + k * FLUSH_ROWS, FLUSH_ROWS)

    mb = [msg_buf0.at[pl.ds(0, FLUSH_ROWS)],
          msg_buf1.at[pl.ds(0, FLUSH_ROWS)],
          msg_buf2.at[pl.ds(0, FLUSH_ROWS)]]
    d0 = pltpu.async_copy(acc_m.at[_mrow(0)], mb[0], aux_sem)
    d1 = pltpu.async_copy(acc_m.at[_mrow(1)], mb[1], aux_sem)
    d2 = pltpu.async_copy(acc_m.at[_mrow(2)], mb[2], aux_sem)
    dv0 = pltpu.async_copy(acc_c.at[pl.ds(row0, RPT)], zbuf_v, aux_sem)
    dv1 = pltpu.async_copy(acc_t.at[pl.ds(row0, RPT)], zbuf_v2, aux_sem)
    d0.wait()
    h0 = pltpu.async_copy(mb[0], psums.at[c, _hrow(0)], sct_sem)
    d1.wait()
    h1 = pltpu.async_copy(mb[1], psums.at[c, _hrow(1)], sct_sem)
    d2.wait()
    h2 = pltpu.async_copy(mb[2], psums.at[c, _hrow(2)], sct_sem)
    dv0.wait()
    hv0 = pltpu.async_copy(zbuf_v, pcnt.at[pl.ds(c * ACC_ROWS + row0, RPT)],
                           sct_sem)
    dv1.wait()
    hv1 = pltpu.async_copy(zbuf_v2, pts.at[pl.ds(c * ACC_ROWS + row0, RPT)],
                           sct_sem)
    h0.wait()
    d3 = pltpu.async_copy(acc_m.at[_mrow(3)], mb[0], aux_sem)
    d3.wait()
    h3 = pltpu.async_copy(mb[0], psums.at[c, _hrow(3)], sct_sem)
    h1.wait()
    h2.wait()
    h3.wait()
    hv0.wait()
    hv1.wait()


def _sc_aggregate(ids_pad, messages, timestamps):
    mesh = plsc.VectorSubcoreMesh(core_axis_name="c", subcore_axis_name="s")
    return pl.kernel(
        _sc_body,
        out_type=[
            jax.ShapeDtypeStruct((NC, ACC_ROWS, D), jnp.float32),
            jax.ShapeDtypeStruct((NC * ACC_ROWS,), jnp.float32),
            jax.ShapeDtypeStruct((NC * ACC_ROWS,), jnp.float32),
        ],
        mesh=mesh,
        scratch_types=[
            pltpu.VMEM((B, D), jnp.float32),
            pltpu.VMEM((B, D), jnp.float32),
            pltpu.VMEM((B, D), jnp.float32),
            pltpu.VMEM((B,), jnp.int32),
            pltpu.VMEM((B,), jnp.int32),
            pltpu.VMEM((B,), jnp.int32),
            pltpu.VMEM((B,), jnp.float32),
            pltpu.VMEM((B,), jnp.float32),
            pltpu.VMEM((B,), jnp.float32),
            pltpu.VMEM((B,), jnp.float32),
            pltpu.VMEM((TPC + K,), jnp.int32),
            pltpu.VMEM((TPC,), jnp.float32),
            pltpu.VMEM((RPT,), jnp.float32),
            pltpu.VMEM((RPT,), jnp.float32),
            pltpu.VMEM_SHARED((ACC_ROWS, D), jnp.float32),
            pltpu.VMEM_SHARED((ACC_ROWS,), jnp.float32),
            pltpu.VMEM_SHARED((ACC_ROWS,), jnp.float32),
            pltpu.SemaphoreType.DMA,
            pltpu.SemaphoreType.DMA,
            pltpu.SemaphoreType.DMA,
        ],
        compiler_params=pltpu.CompilerParams(needs_layout_passes=False),
    )(ids_pad, messages, timestamps)


def _finalize_body(ps_ref, pc_ref, pt_ref, um_ref, ts_ref, msk_ref):
    sums = jnp.concatenate([ps_ref[0, :NPC, :], ps_ref[1, :NPC, :]], axis=0)
    cnt = jnp.concatenate([pc_ref[:NPC], pc_ref[ACC_ROWS:ACC_ROWS + NPC]],
                          axis=0)
    ts = jnp.concatenate([pt_ref[:NPC], pt_ref[ACC_ROWS:ACC_ROWS + NPC]],
                         axis=0)
    um_ref[...] = sums / jnp.maximum(cnt, 1.0)[:, None]
    ts_ref[...] = ts
    msk_ref[...] = (cnt > 0.0).astype(jnp.int32)


def _finalize(psums, pcnt, pts):
    return pl.pallas_call(
        _finalize_body,
        out_shape=[
            jax.ShapeDtypeStruct((N_NODES, D), jnp.float32),
            jax.ShapeDtypeStruct((N_NODES,), jnp.float32),
            jax.ShapeDtypeStruct((N_NODES,), jnp.int32),
        ],
    )(psums, pcnt, pts)


@jax.jit
def kernel(node_ids, messages, timestamps, memory):
    del memory
    ids = node_ids.astype(jnp.int32)
    ids_pad = jnp.concatenate([ids, jnp.full((K,), SENTINEL, jnp.int32)])
    psums, pcnt, pts = _sc_aggregate(ids_pad, messages, timestamps)
    um, ts, msk = _finalize(psums, pcnt, pts)
    return um, ts, msk.astype(bool)
